# baseline (speedup 1.0000x reference)
import functools

import jax
import jax.numpy as jnp
from jax import lax
from jax.experimental import pallas as pl
from jax.experimental.pallas import tpu as pltpu


N_TOK = 8192
G = 4
DIM = 64
CB = 1024
BLK = 1024


def _vq_kernel(x_ref, cb_ref, xh_ref, oh_ref, idx_ref, c2_ref):
    @pl.when(pl.program_id(0) == 0)
    def _():
        cb = cb_ref[...]
        c2_ref[...] = jnp.sum(cb * cb, axis=2)

    idxs = []
    for g in range(G):
        xg = x_ref[g]
        cbg = cb_ref[g]
        neg2s = lax.dot_general(
            xg * (-2.0), cbg,
            dimension_numbers=(((1,), (1,)), ((), ())),
            preferred_element_type=jnp.float32,
        )
        x2 = jnp.sum(xg * xg, axis=1, keepdims=True)
        dist = (x2 + c2_ref[g][None, :]) + neg2s
        idx = jnp.argmin(dist, axis=1).astype(jnp.int32)
        oh = (lax.broadcasted_iota(jnp.int32, (BLK, CB), 1)
              == idx[:, None]).astype(jnp.float32)
        xh = lax.dot_general(
            oh, cbg,
            dimension_numbers=(((1,), (0,)), ((), ())),
            preferred_element_type=jnp.float32,
        )
        xh_ref[:, g, :] = xh
        oh_ref[:, g, :] = oh
        idxs.append(idx)
    idx_ref[...] = jnp.stack(idxs, axis=1)[:, :, None]


@functools.partial(jax.jit, static_argnames=())
def kernel(x, code_book):
    n = x.shape[0]
    xt = x.transpose(1, 0, 2)
    grid = (n // BLK,)
    xh, oh, idx = pl.pallas_call(
        _vq_kernel,
        grid=grid,
        in_specs=[
            pl.BlockSpec((G, BLK, DIM), lambda i: (0, i, 0)),
            pl.BlockSpec((G, CB, DIM), lambda i: (0, 0, 0)),
        ],
        out_specs=[
            pl.BlockSpec((BLK, G, DIM), lambda i: (i, 0, 0)),
            pl.BlockSpec((BLK, G, CB), lambda i: (i, 0, 0)),
            pl.BlockSpec((BLK, G, 1), lambda i: (i, 0, 0)),
        ],
        out_shape=[
            jax.ShapeDtypeStruct((n, G, DIM), jnp.float32),
            jax.ShapeDtypeStruct((n, G, CB), jnp.float32),
            jax.ShapeDtypeStruct((n, G, 1), jnp.int32),
        ],
        scratch_shapes=[pltpu.VMEM((G, CB), jnp.float32)],
    )(xt, code_book)
    return (xh, oh, idx)


# speedup vs baseline: 1.0930x; 1.0930x over previous
import functools

import jax
import jax.numpy as jnp
from jax import lax
from jax.experimental import pallas as pl
from jax.experimental.pallas import tpu as pltpu


N_TOK = 8192
G = 4
DIM = 64
CB = 1024
BLK = 1024


def _vq_kernel(x_ref, cb_ref, xh_ref, oh_ref, idx_ref, c2_ref):
    @pl.when(pl.program_id(0) == 0)
    def _():
        cb = cb_ref[...]
        c2_ref[...] = jnp.sum(cb * cb, axis=2)

    idxs = []
    for g in range(G):
        xg = x_ref[g]
        cbg = cb_ref[g]
        neg2s = lax.dot_general(
            xg * (-2.0), cbg,
            dimension_numbers=(((1,), (1,)), ((), ())),
            preferred_element_type=jnp.float32,
        )
        x2 = jnp.sum(xg * xg, axis=1, keepdims=True)
        dist = (x2 + c2_ref[g][None, :]) + neg2s
        idx = jnp.argmin(dist, axis=1).astype(jnp.int32)
        oh = (lax.broadcasted_iota(jnp.int32, (BLK, CB), 1)
              == idx[:, None]).astype(jnp.float32)
        xh = lax.dot_general(
            oh, cbg,
            dimension_numbers=(((1,), (0,)), ((), ())),
            preferred_element_type=jnp.float32,
        )
        xh_ref[:, g, :] = xh
        idxs.append(idx)
    idx_all = jnp.stack(idxs, axis=1)[:, :, None]
    oh_all = (lax.broadcasted_iota(jnp.int32, (BLK, G, CB), 2)
              == idx_all).astype(jnp.float32)
    idx_ref[...] = idx_all
    oh_ref[...] = oh_all


@functools.partial(jax.jit, static_argnames=())
def kernel(x, code_book):
    n = x.shape[0]
    xt = x.transpose(1, 0, 2)
    grid = (n // BLK,)
    xh, oh, idx = pl.pallas_call(
        _vq_kernel,
        grid=grid,
        in_specs=[
            pl.BlockSpec((G, BLK, DIM), lambda i: (0, i, 0)),
            pl.BlockSpec((G, CB, DIM), lambda i: (0, 0, 0)),
        ],
        out_specs=[
            pl.BlockSpec((BLK, G, DIM), lambda i: (i, 0, 0)),
            pl.BlockSpec((BLK, G, CB), lambda i: (i, 0, 0)),
            pl.BlockSpec((BLK, G, 1), lambda i: (i, 0, 0)),
        ],
        out_shape=[
            jax.ShapeDtypeStruct((n, G, DIM), jnp.float32),
            jax.ShapeDtypeStruct((n, G, CB), jnp.float32),
            jax.ShapeDtypeStruct((n, G, 1), jnp.int32),
        ],
        scratch_shapes=[pltpu.VMEM((G, CB), jnp.float32)],
    )(xt, code_book)
    return (xh, oh, idx)
